# baseline (speedup 1.0000x reference)
import jax
import jax.numpy as jnp
from jax.experimental import pallas as pl
from jax.experimental.pallas import tpu as pltpu

NUM_RAAGS = 1000
EMBED_DIM = 128
SEQ_LEN = 512
BATCH = 1024
B_BLK = 16
NBUF = 4
NSTEP = BATCH // B_BLK


def _tile_kernel(idx_ref, table_ref, out_hbm, scratch, sems):
    i = pl.program_id(0)
    slot = jax.lax.rem(i, NBUF)

    def _copy(step, s):
        return pltpu.make_async_copy(
            scratch.at[s],
            out_hbm.at[pl.ds(step * B_BLK, B_BLK)],
            sems.at[s],
        )

    @pl.when(i >= NBUF)
    def _():
        _copy(i - NBUF, slot).wait()

    for j in range(B_BLK):
        idx = idx_ref[i * B_BLK + j]
        row = table_ref[pl.ds(idx, 1), :]
        scratch[slot, j, :, :] = jnp.broadcast_to(row, (SEQ_LEN, EMBED_DIM))

    _copy(i, slot).start()

    @pl.when(i == NSTEP - 1)
    def _():
        for k in range(NBUF):
            step = NSTEP - NBUF + k
            _copy(step, jax.lax.rem(jnp.int32(step), NBUF)).wait()


def kernel(raag_embeddings, table):
    idx = raag_embeddings.reshape(BATCH)

    grid_spec = pltpu.PrefetchScalarGridSpec(
        num_scalar_prefetch=1,
        grid=(NSTEP,),
        in_specs=[
            pl.BlockSpec((NUM_RAAGS, EMBED_DIM), lambda i, idx_ref: (0, 0)),
        ],
        out_specs=pl.BlockSpec(memory_space=pl.ANY),
        scratch_shapes=[
            pltpu.VMEM((NBUF, B_BLK, SEQ_LEN, EMBED_DIM), jnp.float32),
            pltpu.SemaphoreType.DMA((NBUF,)),
        ],
    )

    out = pl.pallas_call(
        _tile_kernel,
        grid_spec=grid_spec,
        out_shape=jax.ShapeDtypeStruct((BATCH, SEQ_LEN, EMBED_DIM), jnp.float32),
    )(idx, table)
    return out


# speedup vs baseline: 1.0023x; 1.0023x over previous
import jax
import jax.numpy as jnp
from jax.experimental import pallas as pl
from jax.experimental.pallas import tpu as pltpu

NUM_RAAGS = 1000
EMBED_DIM = 128
SEQ_LEN = 512
BATCH = 1024
B_BLK = 8
NBUF = 8
NSTEP = BATCH // B_BLK


def _tile_kernel(idx_ref, table_ref, out_hbm, scratch, sems):
    i = pl.program_id(0)
    slot = jax.lax.rem(i, NBUF)

    def _copy(step, s):
        return pltpu.make_async_copy(
            scratch.at[s],
            out_hbm.at[pl.ds(step * B_BLK, B_BLK)],
            sems.at[s],
        )

    @pl.when(i >= NBUF)
    def _():
        _copy(i - NBUF, slot).wait()

    for j in range(B_BLK):
        idx = idx_ref[i * B_BLK + j]
        row = table_ref[pl.ds(idx, 1), :]
        scratch[slot, j, :, :] = jnp.broadcast_to(row, (SEQ_LEN, EMBED_DIM))

    _copy(i, slot).start()

    @pl.when(i == NSTEP - 1)
    def _():
        for k in range(NBUF):
            step = NSTEP - NBUF + k
            _copy(step, jax.lax.rem(jnp.int32(step), NBUF)).wait()


def kernel(raag_embeddings, table):
    idx = raag_embeddings.reshape(BATCH)

    grid_spec = pltpu.PrefetchScalarGridSpec(
        num_scalar_prefetch=1,
        grid=(NSTEP,),
        in_specs=[
            pl.BlockSpec((NUM_RAAGS, EMBED_DIM), lambda i, idx_ref: (0, 0)),
        ],
        out_specs=pl.BlockSpec(memory_space=pl.ANY),
        scratch_shapes=[
            pltpu.VMEM((NBUF, B_BLK, SEQ_LEN, EMBED_DIM), jnp.float32),
            pltpu.SemaphoreType.DMA((NBUF,)),
        ],
    )

    out = pl.pallas_call(
        _tile_kernel,
        grid_spec=grid_spec,
        out_shape=jax.ShapeDtypeStruct((BATCH, SEQ_LEN, EMBED_DIM), jnp.float32),
    )(idx, table)
    return out


# speedup vs baseline: 1.0195x; 1.0172x over previous
import jax
import jax.numpy as jnp
from jax.experimental import pallas as pl
from jax.experimental.pallas import tpu as pltpu

NUM_RAAGS = 1000
EMBED_DIM = 128
SEQ_LEN = 512
BATCH = 1024
B_BLK = 16


def _tile_kernel(idx_ref, table_ref, out_ref):
    i = pl.program_id(0)
    for j in range(B_BLK):
        idx = idx_ref[i * B_BLK + j]
        row = table_ref[pl.ds(idx, 1), :]
        out_ref[j, :, :] = jnp.broadcast_to(row, (SEQ_LEN, EMBED_DIM))


def kernel(raag_embeddings, table):
    idx = raag_embeddings.reshape(BATCH)

    grid_spec = pltpu.PrefetchScalarGridSpec(
        num_scalar_prefetch=1,
        grid=(BATCH // B_BLK,),
        in_specs=[
            pl.BlockSpec((NUM_RAAGS, EMBED_DIM), lambda i, idx_ref: (0, 0)),
        ],
        out_specs=pl.BlockSpec(
            (B_BLK, SEQ_LEN, EMBED_DIM), lambda i, idx_ref: (i, 0, 0)
        ),
    )

    out = pl.pallas_call(
        _tile_kernel,
        grid_spec=grid_spec,
        out_shape=jax.ShapeDtypeStruct((BATCH, SEQ_LEN, EMBED_DIM), jnp.float32),
    )(idx, table)
    return out
